# baseline (speedup 1.0000x reference)
import functools

import jax
import jax.numpy as jnp
from jax import lax
from jax.experimental import pallas as pl
from jax.experimental.pallas import tpu as pltpu
from jax.experimental.pallas import tpu_sc as plsc

_EPS = 1e-6


def _gather_sc(x2, table):
    N = x2.shape[0] * x2.shape[1]
    V, D = table.shape
    info = plsc.get_sparse_core_info()
    NC, NS = info.num_cores, info.num_subcores
    NW = NC * NS
    K = 128
    per_w = N // NW
    n_chunks = per_w // K
    mesh = plsc.VectorSubcoreMesh(core_axis_name="c", subcore_axis_name="s")

    @functools.partial(
        pl.kernel,
        mesh=mesh,
        out_type=jax.ShapeDtypeStruct((N, D), jnp.float32),
        scratch_types=[
            pltpu.VMEM((n_chunks, K), jnp.int32),
            pltpu.VMEM((2, K, D), jnp.float32),
            pltpu.VMEM_SHARED((V, D), jnp.float32),
            pltpu.SemaphoreType.DMA,
            pltpu.SemaphoreType.DMA,
            pltpu.SemaphoreType.DMA,
            pltpu.SemaphoreType.DMA,
        ],
    )
    def k(x_hbm, tbl_hbm, out_hbm, idx_v, rows_v, tbl_sp, g0, g1, s0, s1):
        sid = lax.axis_index("s")
        wid = sid * NC + lax.axis_index("c")
        base = wid * per_w
        gsem = (g0, g1)
        ssem = (s0, s1)
        vmain = (V // NS) * NS
        pltpu.sync_copy(tbl_hbm.at[pl.ds(sid * (vmain // NS), vmain // NS)],
                        tbl_sp.at[pl.ds(sid * (vmain // NS), vmain // NS)])

        @pl.when(sid == 0)
        def _():
            pltpu.sync_copy(tbl_hbm.at[pl.ds(vmain, V - vmain)],
                            tbl_sp.at[pl.ds(vmain, V - vmain)])

        pltpu.sync_copy(x_hbm.at[pl.ds(wid * n_chunks, n_chunks)], idx_v)
        plsc.subcore_barrier()
        hg = [None] * n_chunks
        hs = [None] * n_chunks
        hg[0] = pltpu.async_copy(
            tbl_sp.at[idx_v.at[0]], rows_v.at[0], gsem[0])
        for c in range(n_chunks):
            buf = c % 2
            if c >= 1:
                hs[c - 1].wait()
            if c + 1 < n_chunks:
                nb = (c + 1) % 2
                hg[c + 1] = pltpu.async_copy(
                    tbl_sp.at[idx_v.at[c + 1]],
                    rows_v.at[nb], gsem[nb])
            hg[c].wait()
            hs[c] = pltpu.async_copy(
                rows_v.at[buf], out_hbm.at[pl.ds(base + c * K, K)], ssem[buf])
        hs[n_chunks - 1].wait()

    return k(x2, table)


def _ln(h, g, b):
    mean = jnp.mean(h, axis=-1, keepdims=True)
    d = h - mean
    var = jnp.mean(d * d, axis=-1, keepdims=True)
    return g * (d * lax.rsqrt(var + _EPS)) + b


def _table_ln_tc(table, gamma, beta, x):
    V, D = table.shape
    B, S = x.shape
    BLK = 2048
    NB = pl.cdiv(V, BLK)

    def body(t_ref, x_ref, g_ref, b_ref, out_ref, x2_ref):
        out_ref[...] = _ln(t_ref[...], g_ref[0], b_ref[0])

        @pl.when(pl.program_id(0) == 0)
        def _():
            x2_ref[...] = x_ref[...].reshape(B * S // D, D)

    vec_spec = pl.BlockSpec((1, D), lambda i: (0, 0))
    row_spec = pl.BlockSpec((BLK, D), lambda i: (i, 0))
    x_spec = pl.BlockSpec((B, S), lambda i: (0, 0))
    x2_spec = pl.BlockSpec((B * S // D, D), lambda i: (0, 0))
    return pl.pallas_call(
        body,
        grid=(NB,),
        in_specs=[row_spec, x_spec, vec_spec, vec_spec],
        out_specs=[row_spec, x2_spec],
        out_shape=[
            jax.ShapeDtypeStruct((V, D), jnp.float32),
            jax.ShapeDtypeStruct((B * S // D, D), jnp.int32),
        ],
    )(table, x, gamma.reshape(1, D), beta.reshape(1, D))


def _erpm_tc(er_embed, pm_embed, gamma, beta, B):
    R, D = er_embed.shape
    NB = 4
    RBLK = R // NB

    def body(er_ref, pm_ref, g_ref, b_ref, er_out_ref, pm_out_ref):
        g = g_ref[0]
        b = b_ref[0]
        for src, dst in ((er_ref, er_out_ref), (pm_ref, pm_out_ref)):
            y = _ln(src[...], g, b)
            dst[...] = jnp.broadcast_to(y[None], (B, RBLK, D))

    vec_spec = pl.BlockSpec((1, D), lambda i: (0, 0))
    row_spec = pl.BlockSpec((RBLK, D), lambda i: (i, 0))
    out_spec = pl.BlockSpec((B, RBLK, D), lambda i: (0, i, 0))
    return pl.pallas_call(
        body,
        grid=(NB,),
        in_specs=[row_spec, row_spec, vec_spec, vec_spec],
        out_specs=[out_spec, out_spec],
        out_shape=[
            jax.ShapeDtypeStruct((B, R, D), jnp.float32),
            jax.ShapeDtypeStruct((B, R, D), jnp.float32),
        ],
    )(er_embed, pm_embed, gamma.reshape(1, D), beta.reshape(1, D))


def kernel(x, er_embed, pm_embed, token_table, pos_table, gamma, beta):
    B, S = x.shape
    D = token_table.shape[1]
    tbl_norm, x2 = _table_ln_tc(token_table, gamma, beta, x)
    token_flat = _gather_sc(x2, tbl_norm)
    er4, pm4 = _erpm_tc(er_embed, pm_embed, gamma, beta, B)
    return token_flat.reshape(B, S, D), er4, pm4


# speedup vs baseline: 1.0231x; 1.0231x over previous
import functools

import jax
import jax.numpy as jnp
from jax import lax
from jax.experimental import pallas as pl
from jax.experimental.pallas import tpu as pltpu
from jax.experimental.pallas import tpu_sc as plsc

_EPS = 1e-6


def _gather_sc(x2, table):
    N = x2.shape[0] * x2.shape[1]
    V, D = table.shape
    info = plsc.get_sparse_core_info()
    NC, NS = info.num_cores, info.num_subcores
    NW = NC * NS
    K = 128
    per_w = N // NW
    n_chunks = per_w // K
    mesh = plsc.VectorSubcoreMesh(core_axis_name="c", subcore_axis_name="s")

    @functools.partial(
        pl.kernel,
        mesh=mesh,
        out_type=jax.ShapeDtypeStruct((N, D), jnp.float32),
        scratch_types=[
            pltpu.VMEM((n_chunks, K), jnp.int32),
            pltpu.VMEM((2, K, D), jnp.float32),
            pltpu.VMEM_SHARED((V, D), jnp.float32),
            pltpu.SemaphoreType.DMA,
            pltpu.SemaphoreType.DMA,
            pltpu.SemaphoreType.DMA,
            pltpu.SemaphoreType.DMA,
        ],
    )
    def k(x_hbm, tbl_hbm, out_hbm, idx_v, rows_v, tbl_sp, g0, g1, s0, s1):
        sid = lax.axis_index("s")
        wid = sid * NC + lax.axis_index("c")
        base = wid * per_w
        gsem = (g0, g1)
        ssem = (s0, s1)
        vmain = (V // NS) * NS
        pltpu.sync_copy(tbl_hbm.at[pl.ds(sid * (vmain // NS), vmain // NS)],
                        tbl_sp.at[pl.ds(sid * (vmain // NS), vmain // NS)])

        @pl.when(sid == 0)
        def _():
            pltpu.sync_copy(tbl_hbm.at[pl.ds(vmain, V - vmain)],
                            tbl_sp.at[pl.ds(vmain, V - vmain)])

        pltpu.sync_copy(x_hbm.at[pl.ds(wid * n_chunks, n_chunks)], idx_v)
        plsc.subcore_barrier()
        hg = [None] * n_chunks
        hs = [None] * n_chunks
        hg[0] = pltpu.async_copy(
            tbl_sp.at[idx_v.at[0]], rows_v.at[0], gsem[0])
        for c in range(n_chunks):
            buf = c % 2
            if c >= 1:
                hs[c - 1].wait()
            if c + 1 < n_chunks:
                nb = (c + 1) % 2
                hg[c + 1] = pltpu.async_copy(
                    tbl_sp.at[idx_v.at[c + 1]],
                    rows_v.at[nb], gsem[nb])
            hg[c].wait()
            hs[c] = pltpu.async_copy(
                rows_v.at[buf], out_hbm.at[pl.ds(base + c * K, K)], ssem[buf])
        hs[n_chunks - 1].wait()

    return k(x2, table)


def _ln(h, g, b):
    mean = jnp.mean(h, axis=-1, keepdims=True)
    d = h - mean
    var = jnp.mean(d * d, axis=-1, keepdims=True)
    return g * (d * lax.rsqrt(var + _EPS)) + b


def _table_ln_tc(table, gamma, beta, x):
    V, D = table.shape
    B, S = x.shape
    BLK = 4096
    NB = pl.cdiv(V, BLK)

    def body(t_ref, x_ref, g_ref, b_ref, out_ref, x2_ref):
        out_ref[...] = _ln(t_ref[...], g_ref[0], b_ref[0])

        @pl.when(pl.program_id(0) == 0)
        def _():
            x2_ref[...] = x_ref[...].reshape(B * S // D, D)

    vec_spec = pl.BlockSpec((1, D), lambda i: (0, 0))
    row_spec = pl.BlockSpec((BLK, D), lambda i: (i, 0))
    x_spec = pl.BlockSpec((B, S), lambda i: (0, 0))
    x2_spec = pl.BlockSpec((B * S // D, D), lambda i: (0, 0))
    return pl.pallas_call(
        body,
        grid=(NB,),
        in_specs=[row_spec, x_spec, vec_spec, vec_spec],
        out_specs=[row_spec, x2_spec],
        out_shape=[
            jax.ShapeDtypeStruct((V, D), jnp.float32),
            jax.ShapeDtypeStruct((B * S // D, D), jnp.int32),
        ],
    )(table, x, gamma.reshape(1, D), beta.reshape(1, D))


def _erpm_tc(er_embed, pm_embed, gamma, beta, B):
    R, D = er_embed.shape
    NB = 2
    RBLK = R // NB

    def body(er_ref, pm_ref, g_ref, b_ref, er_out_ref, pm_out_ref):
        g = g_ref[0]
        b = b_ref[0]
        for src, dst in ((er_ref, er_out_ref), (pm_ref, pm_out_ref)):
            y = _ln(src[...], g, b)
            dst[...] = jnp.broadcast_to(y[None], (B, RBLK, D))

    vec_spec = pl.BlockSpec((1, D), lambda i: (0, 0))
    row_spec = pl.BlockSpec((RBLK, D), lambda i: (i, 0))
    out_spec = pl.BlockSpec((B, RBLK, D), lambda i: (0, i, 0))
    return pl.pallas_call(
        body,
        grid=(NB,),
        in_specs=[row_spec, row_spec, vec_spec, vec_spec],
        out_specs=[out_spec, out_spec],
        out_shape=[
            jax.ShapeDtypeStruct((B, R, D), jnp.float32),
            jax.ShapeDtypeStruct((B, R, D), jnp.float32),
        ],
    )(er_embed, pm_embed, gamma.reshape(1, D), beta.reshape(1, D))


def kernel(x, er_embed, pm_embed, token_table, pos_table, gamma, beta):
    B, S = x.shape
    D = token_table.shape[1]
    tbl_norm, x2 = _table_ln_tc(token_table, gamma, beta, x)
    token_flat = _gather_sc(x2, tbl_norm)
    er4, pm4 = _erpm_tc(er_embed, pm_embed, gamma, beta, B)
    return token_flat.reshape(B, S, D), er4, pm4


# speedup vs baseline: 1.0255x; 1.0023x over previous
import functools

import jax
import jax.numpy as jnp
from jax import lax
from jax.experimental import pallas as pl
from jax.experimental.pallas import tpu as pltpu
from jax.experimental.pallas import tpu_sc as plsc

_EPS = 1e-6


def _gather_sc(x2, table):
    N = x2.shape[0] * x2.shape[1]
    V, D = table.shape
    info = plsc.get_sparse_core_info()
    NC, NS = info.num_cores, info.num_subcores
    NW = NC * NS
    K = 128
    per_w = N // NW
    n_chunks = per_w // K
    mesh = plsc.VectorSubcoreMesh(core_axis_name="c", subcore_axis_name="s")

    @functools.partial(
        pl.kernel,
        mesh=mesh,
        out_type=jax.ShapeDtypeStruct((N, D), jnp.float32),
        scratch_types=[
            pltpu.VMEM((n_chunks, K), jnp.int32),
            pltpu.VMEM((3, K, D), jnp.float32),
            pltpu.VMEM_SHARED((V, D), jnp.float32),
            pltpu.SemaphoreType.DMA,
            pltpu.SemaphoreType.DMA,
            pltpu.SemaphoreType.DMA,
            pltpu.SemaphoreType.DMA,
            pltpu.SemaphoreType.DMA,
            pltpu.SemaphoreType.DMA,
        ],
    )
    def k(x_hbm, tbl_hbm, out_hbm, idx_v, rows_v, tbl_sp,
          g0, g1, g2, s0, s1, s2):
        sid = lax.axis_index("s")
        wid = sid * NC + lax.axis_index("c")
        base = wid * per_w
        gsem = (g0, g1, g2)
        ssem = (s0, s1, s2)
        vmain = (V // NS) * NS
        pltpu.sync_copy(tbl_hbm.at[pl.ds(sid * (vmain // NS), vmain // NS)],
                        tbl_sp.at[pl.ds(sid * (vmain // NS), vmain // NS)])

        @pl.when(sid == 0)
        def _():
            pltpu.sync_copy(tbl_hbm.at[pl.ds(vmain, V - vmain)],
                            tbl_sp.at[pl.ds(vmain, V - vmain)])

        pltpu.sync_copy(x_hbm.at[pl.ds(wid * n_chunks, n_chunks)], idx_v)
        plsc.subcore_barrier()
        NBUF = 3
        hg = [None] * n_chunks
        hs = [None] * n_chunks
        hg[0] = pltpu.async_copy(
            tbl_sp.at[idx_v.at[0]], rows_v.at[0], gsem[0])
        for c in range(n_chunks):
            buf = c % NBUF
            if c + 1 < n_chunks:
                nb = (c + 1) % NBUF
                if c + 1 >= NBUF:
                    hs[c + 1 - NBUF].wait()
                hg[c + 1] = pltpu.async_copy(
                    tbl_sp.at[idx_v.at[c + 1]],
                    rows_v.at[nb], gsem[nb])
            hg[c].wait()
            hs[c] = pltpu.async_copy(
                rows_v.at[buf], out_hbm.at[pl.ds(base + c * K, K)], ssem[buf])
        for c in range(max(0, n_chunks - NBUF), n_chunks):
            hs[c].wait()

    return k(x2, table)


def _ln(h, g, b):
    mean = jnp.mean(h, axis=-1, keepdims=True)
    d = h - mean
    var = jnp.mean(d * d, axis=-1, keepdims=True)
    return g * (d * lax.rsqrt(var + _EPS)) + b


def _table_ln_tc(table, gamma, beta, x):
    V, D = table.shape
    B, S = x.shape
    BLK = 4096
    NB = pl.cdiv(V, BLK)

    def body(t_ref, x_ref, g_ref, b_ref, out_ref, x2_ref):
        out_ref[...] = _ln(t_ref[...], g_ref[0], b_ref[0])

        @pl.when(pl.program_id(0) == 0)
        def _():
            x2_ref[...] = x_ref[...].reshape(B * S // D, D)

    vec_spec = pl.BlockSpec((1, D), lambda i: (0, 0))
    row_spec = pl.BlockSpec((BLK, D), lambda i: (i, 0))
    x_spec = pl.BlockSpec((B, S), lambda i: (0, 0))
    x2_spec = pl.BlockSpec((B * S // D, D), lambda i: (0, 0))
    return pl.pallas_call(
        body,
        grid=(NB,),
        in_specs=[row_spec, x_spec, vec_spec, vec_spec],
        out_specs=[row_spec, x2_spec],
        out_shape=[
            jax.ShapeDtypeStruct((V, D), jnp.float32),
            jax.ShapeDtypeStruct((B * S // D, D), jnp.int32),
        ],
    )(table, x, gamma.reshape(1, D), beta.reshape(1, D))


def _erpm_tc(er_embed, pm_embed, gamma, beta, B):
    R, D = er_embed.shape
    NB = 2
    RBLK = R // NB

    def body(er_ref, pm_ref, g_ref, b_ref, er_out_ref, pm_out_ref):
        g = g_ref[0]
        b = b_ref[0]
        for src, dst in ((er_ref, er_out_ref), (pm_ref, pm_out_ref)):
            y = _ln(src[...], g, b)
            dst[...] = jnp.broadcast_to(y[None], (B, RBLK, D))

    vec_spec = pl.BlockSpec((1, D), lambda i: (0, 0))
    row_spec = pl.BlockSpec((RBLK, D), lambda i: (i, 0))
    out_spec = pl.BlockSpec((B, RBLK, D), lambda i: (0, i, 0))
    return pl.pallas_call(
        body,
        grid=(NB,),
        in_specs=[row_spec, row_spec, vec_spec, vec_spec],
        out_specs=[out_spec, out_spec],
        out_shape=[
            jax.ShapeDtypeStruct((B, R, D), jnp.float32),
            jax.ShapeDtypeStruct((B, R, D), jnp.float32),
        ],
    )(er_embed, pm_embed, gamma.reshape(1, D), beta.reshape(1, D))


def kernel(x, er_embed, pm_embed, token_table, pos_table, gamma, beta):
    B, S = x.shape
    D = token_table.shape[1]
    tbl_norm, x2 = _table_ln_tc(token_table, gamma, beta, x)
    token_flat = _gather_sc(x2, tbl_norm)
    er4, pm4 = _erpm_tc(er_embed, pm_embed, gamma, beta, B)
    return token_flat.reshape(B, S, D), er4, pm4
